# baseline (speedup 1.0000x reference)
import functools

import jax
import jax.numpy as jnp
from jax import lax
from jax.experimental import pallas as pl
from jax.experimental.pallas import tpu as pltpu
from jax.experimental.pallas import tpu_sc as plsc

VOCAB = 50
D = 16
B = 16384
L = 200
N = B * L
NC = 2
NS = 16
NW = NC * NS
IDXW = N // NW
CHUNK = 2048
NCHUNK = IDXW // CHUNK
GROUPS = CHUNK // 16


def _body(tags_ref, table_ref, out_ref, tab_v, idx_v0, idx_v1, out_v0, out_v1,
          si0, si1, so0, so1):
    wid = lax.axis_index("s") * NC + lax.axis_index("c")
    si = (si0, si1)
    so = (so0, so1)
    idx_v = (idx_v0, idx_v1)
    out_v = (out_v0, out_v1)

    pltpu.sync_copy(table_ref, tab_v)
    for i in range(VOCAB):
        tab_v[pl.ds(i * D, D)] = jnp.maximum(tab_v[pl.ds(i * D, D)], 0.0)

    lane = lax.iota(jnp.int32, 16)
    row_off = lane * D

    def ibase(c):
        return wid * IDXW + c * CHUNK

    def issue_idx(c, b):
        pltpu.async_copy(tags_ref.at[pl.ds(ibase(c), CHUNK)], idx_v[b], si[b])

    def wait_idx(b):
        pltpu.make_async_copy(tags_ref.at[pl.ds(0, CHUNK)], idx_v[b], si[b]).wait()

    def issue_out(c, b):
        pltpu.async_copy(out_v[b], out_ref.at[pl.ds(ibase(c) * D, CHUNK * D)], so[b])

    def wait_out(b):
        pltpu.make_async_copy(out_v[b], out_ref.at[pl.ds(0, CHUNK * D)], so[b]).wait()

    issue_idx(0, 0)
    issue_idx(1, 1)

    @pl.loop(0, NCHUNK // 2)
    def _super(s):
        for b in range(2):
            c = s * 2 + b

            wait_idx(b)

            @pl.when(c >= 2)
            def _():
                wait_out(b)

            @pl.loop(0, GROUPS, unroll=4)
            def _grp(g):
                iv = idx_v[b][pl.ds(g * 16, 16)]
                addr = iv * D
                dst = g * (16 * D) + row_off
                for d in range(D):
                    vals = plsc.load_gather(tab_v, [addr + d])
                    plsc.store_scatter(out_v[b], [dst + d], vals)

            issue_out(c, b)

            @pl.when(c + 2 < NCHUNK)
            def _():
                issue_idx(c + 2, b)

    wait_out(0)
    wait_out(1)


@jax.jit
def _run(tags1d, table1d):
    mesh = plsc.VectorSubcoreMesh(
        core_axis_name="c", subcore_axis_name="s", num_cores=NC, num_subcores=NS
    )
    kern = pl.kernel(
        _body,
        out_type=jax.ShapeDtypeStruct((N * D,), jnp.float32),
        mesh=mesh,
        scratch_types=[
            pltpu.VMEM((VOCAB * D,), jnp.float32),
            pltpu.VMEM((CHUNK,), jnp.int32),
            pltpu.VMEM((CHUNK,), jnp.int32),
            pltpu.VMEM((CHUNK * D,), jnp.float32),
            pltpu.VMEM((CHUNK * D,), jnp.float32),
            pltpu.SemaphoreType.DMA,
            pltpu.SemaphoreType.DMA,
            pltpu.SemaphoreType.DMA,
            pltpu.SemaphoreType.DMA,
        ],
        compiler_params=pltpu.CompilerParams(needs_layout_passes=False),
    )
    return kern(tags1d, table1d)


def kernel(preprocessed_tags, embedding):
    tags1d = preprocessed_tags.reshape(N)
    table1d = embedding.reshape(VOCAB * D)
    out = _run(tags1d, table1d)
    return out.reshape(B, L, D)


# speedup vs baseline: 1.1029x; 1.1029x over previous
import functools

import jax
import jax.numpy as jnp
from jax import lax
from jax.experimental import pallas as pl
from jax.experimental.pallas import tpu as pltpu
from jax.experimental.pallas import tpu_sc as plsc

VOCAB = 50
D = 16
B = 16384
L = 200
N = B * L
NC = 2
NS = 16
NW = NC * NS
IDXW = N // NW
CHUNK = 2048
NCHUNK = IDXW // CHUNK
GROUPS = CHUNK // 16


def _body(tags_ref, table_ref, out_ref, tab_v, idx_v0, idx_v1, out_v0, out_v1,
          si0, si1, so0, so1):
    wid = lax.axis_index("s") * NC + lax.axis_index("c")
    si = (si0, si1)
    so = (so0, so1)
    idx_v = (idx_v0, idx_v1)
    out_v = (out_v0, out_v1)

    pltpu.sync_copy(table_ref, tab_v)
    for i in range(VOCAB):
        tab_v[pl.ds(i * D, D)] = jnp.maximum(tab_v[pl.ds(i * D, D)], 0.0)

    lane = lax.iota(jnp.int32, 16)
    row_off = lane * D

    def ibase(c):
        return wid * IDXW + c * CHUNK

    def issue_idx(c, b):
        pltpu.async_copy(tags_ref.at[pl.ds(ibase(c), CHUNK)], idx_v[b], si[b])

    def wait_idx(b):
        pltpu.make_async_copy(tags_ref.at[pl.ds(0, CHUNK)], idx_v[b], si[b]).wait()

    def issue_out(c, b):
        pltpu.async_copy(out_v[b], out_ref.at[pl.ds(ibase(c) * D, CHUNK * D)], so[b])

    def wait_out(b):
        pltpu.make_async_copy(out_v[b], out_ref.at[pl.ds(0, CHUNK * D)], so[b]).wait()

    issue_idx(0, 0)
    issue_idx(1, 1)

    @pl.loop(0, NCHUNK // 2)
    def _super(s):
        for b in range(2):
            c = s * 2 + b

            wait_idx(b)

            @pl.when(c >= 2)
            def _():
                wait_out(b)

            @plsc.parallel_loop(0, GROUPS, unroll=4)
            def _grp(g):
                iv = idx_v[b][pl.ds(g * 16, 16)]
                addr = iv * D
                dst = g * (16 * D) + row_off
                for d in range(D):
                    vals = plsc.load_gather(tab_v, [addr + d])
                    plsc.store_scatter(out_v[b], [dst + d], vals)

            issue_out(c, b)

            @pl.when(c + 2 < NCHUNK)
            def _():
                issue_idx(c + 2, b)

    wait_out(0)
    wait_out(1)


@jax.jit
def _run(tags1d, table1d):
    mesh = plsc.VectorSubcoreMesh(
        core_axis_name="c", subcore_axis_name="s", num_cores=NC, num_subcores=NS
    )
    kern = pl.kernel(
        _body,
        out_type=jax.ShapeDtypeStruct((N * D,), jnp.float32),
        mesh=mesh,
        scratch_types=[
            pltpu.VMEM((VOCAB * D,), jnp.float32),
            pltpu.VMEM((CHUNK,), jnp.int32),
            pltpu.VMEM((CHUNK,), jnp.int32),
            pltpu.VMEM((CHUNK * D,), jnp.float32),
            pltpu.VMEM((CHUNK * D,), jnp.float32),
            pltpu.SemaphoreType.DMA,
            pltpu.SemaphoreType.DMA,
            pltpu.SemaphoreType.DMA,
            pltpu.SemaphoreType.DMA,
        ],
        compiler_params=pltpu.CompilerParams(needs_layout_passes=False),
    )
    return kern(tags1d, table1d)


def kernel(preprocessed_tags, embedding):
    tags1d = preprocessed_tags.reshape(N)
    table1d = embedding.reshape(VOCAB * D)
    out = _run(tags1d, table1d)
    return out.reshape(B, L, D)


# speedup vs baseline: 1.1124x; 1.0085x over previous
import functools

import jax
import jax.numpy as jnp
from jax import lax
from jax.experimental import pallas as pl
from jax.experimental.pallas import tpu as pltpu
from jax.experimental.pallas import tpu_sc as plsc

VOCAB = 50
D = 16
B = 16384
L = 200
N = B * L
NC = 2
NS = 16
NW = NC * NS
IDXW = N // NW
CHUNK = 3200
NCHUNK = IDXW // CHUNK
GROUPS = CHUNK // 16


def _body(tags_ref, table_ref, out_ref, tab_v, idx_v0, idx_v1, out_v0, out_v1,
          si0, si1, so0, so1):
    wid = lax.axis_index("s") * NC + lax.axis_index("c")
    si = (si0, si1)
    so = (so0, so1)
    idx_v = (idx_v0, idx_v1)
    out_v = (out_v0, out_v1)

    pltpu.sync_copy(table_ref, tab_v)
    for i in range(VOCAB):
        tab_v[pl.ds(i * D, D)] = jnp.maximum(tab_v[pl.ds(i * D, D)], 0.0)

    lane = lax.iota(jnp.int32, 16)
    row_off = lane * D

    def ibase(c):
        return wid * IDXW + c * CHUNK

    def issue_idx(c, b):
        pltpu.async_copy(tags_ref.at[pl.ds(ibase(c), CHUNK)], idx_v[b], si[b])

    def wait_idx(b):
        pltpu.make_async_copy(tags_ref.at[pl.ds(0, CHUNK)], idx_v[b], si[b]).wait()

    def issue_out(c, b):
        pltpu.async_copy(out_v[b], out_ref.at[pl.ds(ibase(c) * D, CHUNK * D)], so[b])

    def wait_out(b):
        pltpu.make_async_copy(out_v[b], out_ref.at[pl.ds(0, CHUNK * D)], so[b]).wait()

    issue_idx(0, 0)
    issue_idx(1, 1)

    @pl.loop(0, NCHUNK // 2)
    def _super(s):
        for b in range(2):
            c = s * 2 + b

            wait_idx(b)

            @pl.when(c >= 2)
            def _():
                wait_out(b)

            @plsc.parallel_loop(0, GROUPS, unroll=8)
            def _grp(g):
                iv = idx_v[b][pl.ds(g * 16, 16)]
                addr = iv * D
                dst = g * (16 * D) + row_off
                for d in range(D):
                    vals = plsc.load_gather(tab_v, [addr + d])
                    plsc.store_scatter(out_v[b], [dst + d], vals)

            issue_out(c, b)

            @pl.when(c + 2 < NCHUNK)
            def _():
                issue_idx(c + 2, b)

    wait_out(0)
    wait_out(1)


@jax.jit
def _run(tags1d, table1d):
    mesh = plsc.VectorSubcoreMesh(
        core_axis_name="c", subcore_axis_name="s", num_cores=NC, num_subcores=NS
    )
    kern = pl.kernel(
        _body,
        out_type=jax.ShapeDtypeStruct((N * D,), jnp.float32),
        mesh=mesh,
        scratch_types=[
            pltpu.VMEM((VOCAB * D,), jnp.float32),
            pltpu.VMEM((CHUNK,), jnp.int32),
            pltpu.VMEM((CHUNK,), jnp.int32),
            pltpu.VMEM((CHUNK * D,), jnp.float32),
            pltpu.VMEM((CHUNK * D,), jnp.float32),
            pltpu.SemaphoreType.DMA,
            pltpu.SemaphoreType.DMA,
            pltpu.SemaphoreType.DMA,
            pltpu.SemaphoreType.DMA,
        ],
        compiler_params=pltpu.CompilerParams(needs_layout_passes=False),
    )
    return kern(tags1d, table1d)


def kernel(preprocessed_tags, embedding):
    tags1d = preprocessed_tags.reshape(N)
    table1d = embedding.reshape(VOCAB * D)
    out = _run(tags1d, table1d)
    return out.reshape(B, L, D)
